# Initial kernel scaffold; baseline (speedup 1.0000x reference)
import jax
import jax.numpy as jnp
from jax.experimental import pallas as pl


def kernel(x, edge_index, W_self0, W_neigh0, b0, W_self1, W_neigh1, b1, W_self2, W_neigh2, b2, W_cls, b_cls):
    raise NotImplementedError("write your pallas kernel here")


# speedup vs baseline: 3.2822x; 3.2822x over previous
import functools

import jax
import jax.numpy as jnp
from jax import lax
from jax.experimental import pallas as pl
from jax.experimental.pallas import tpu as pltpu
from jax.experimental.pallas import tpu_sc as plsc

N = 10000
E = 320000
D = 128
NCLS = 10

NC = 2
NS = 16
NW = NC * NS

CHUNK = 128
CPT = 80
NCHUNKS = NW * CPT
EPAD = NCHUNKS * CHUNK
DUMMY = N

BLK = 1024
NPAD = 10240
RPT = NPAD // NS
GRID = NPAD // BLK

_mesh = plsc.VectorSubcoreMesh(core_axis_name="c", subcore_axis_name="s")


def _sc_agg_deg_body(h_hbm, src_hbm, dst_hbm, z2_hbm, z1_hbm,
                     out_hbm, deg_hbm,
                     srcb, dstb, rows, onesb, acc, degacc):
    c = lax.axis_index("c")
    s = lax.axis_index("s")
    wid = c * NS + s
    pltpu.sync_copy(z2_hbm.at[pl.ds(s * RPT, RPT)], acc.at[pl.ds(s * RPT, RPT)])
    pltpu.sync_copy(z1_hbm.at[pl.ds(s * RPT, RPT)], degacc.at[pl.ds(s * RPT, RPT)])
    for i in range(CHUNK // 16):
        onesb[pl.ds(i * 16, 16)] = jnp.ones((16,), jnp.float32)
    pltpu.sync_copy(src_hbm.at[pl.ds(wid * CPT * CHUNK, CPT * CHUNK)], srcb)
    pltpu.sync_copy(dst_hbm.at[pl.ds(wid * CPT, CPT)], dstb)
    plsc.subcore_barrier()

    def step(j, carry):
        pltpu.sync_copy(h_hbm.at[srcb.at[pl.ds(j * CHUNK, CHUNK)]], rows)
        pltpu.sync_copy(rows, acc.at[dstb.at[j]], add=True)
        pltpu.sync_copy(onesb, degacc.at[dstb.at[j]], add=True)
        return carry

    lax.fori_loop(0, CPT, step, 0)
    plsc.subcore_barrier()
    pltpu.sync_copy(acc.at[pl.ds(s * RPT, RPT)],
                    out_hbm.at[pl.ds(c * NPAD + s * RPT, RPT)])
    pltpu.sync_copy(degacc.at[pl.ds(s * RPT, RPT)],
                    deg_hbm.at[pl.ds(c * NPAD + s * RPT, RPT)])


def _sc_agg_body(h_hbm, src_hbm, dst_hbm, z2_hbm,
                 out_hbm,
                 srcb, dstb, rows, acc):
    c = lax.axis_index("c")
    s = lax.axis_index("s")
    wid = c * NS + s
    pltpu.sync_copy(z2_hbm.at[pl.ds(s * RPT, RPT)], acc.at[pl.ds(s * RPT, RPT)])
    pltpu.sync_copy(src_hbm.at[pl.ds(wid * CPT * CHUNK, CPT * CHUNK)], srcb)
    pltpu.sync_copy(dst_hbm.at[pl.ds(wid * CPT, CPT)], dstb)
    plsc.subcore_barrier()

    def step(j, carry):
        pltpu.sync_copy(h_hbm.at[srcb.at[pl.ds(j * CHUNK, CHUNK)]], rows)
        pltpu.sync_copy(rows, acc.at[dstb.at[j]], add=True)
        return carry

    lax.fori_loop(0, CPT, step, 0)
    plsc.subcore_barrier()
    pltpu.sync_copy(acc.at[pl.ds(s * RPT, RPT)],
                    out_hbm.at[pl.ds(c * NPAD + s * RPT, RPT)])


_sc_agg_deg = pl.kernel(
    _sc_agg_deg_body,
    out_type=(jax.ShapeDtypeStruct((NC * NPAD, D), jnp.float32),
              jax.ShapeDtypeStruct((NC * NPAD,), jnp.float32)),
    mesh=_mesh,
    scratch_types=[
        pltpu.VMEM((CPT * CHUNK,), jnp.int32),
        pltpu.VMEM((CPT, CHUNK), jnp.int32),
        pltpu.VMEM((CHUNK, D), jnp.float32),
        pltpu.VMEM((CHUNK,), jnp.float32),
        pltpu.VMEM_SHARED((NPAD, D), jnp.float32),
        pltpu.VMEM_SHARED((NPAD,), jnp.float32),
    ],
)

_sc_agg = pl.kernel(
    _sc_agg_body,
    out_type=jax.ShapeDtypeStruct((NC * NPAD, D), jnp.float32),
    mesh=_mesh,
    scratch_types=[
        pltpu.VMEM((CPT * CHUNK,), jnp.int32),
        pltpu.VMEM((CPT, CHUNK), jnp.int32),
        pltpu.VMEM((CHUNK, D), jnp.float32),
        pltpu.VMEM_SHARED((NPAD, D), jnp.float32),
    ],
)


def _combine_body(h_ref, s0_ref, s1_ref, d0_ref, d1_ref,
                  ws_ref, wn_ref, b_ref, o_ref, *, readout):
    deg = d0_ref[...] + d1_ref[...]
    inv = 1.0 / jnp.maximum(deg, 1.0)
    hn = (s0_ref[...] + s1_ref[...]) * inv
    acc = jnp.dot(h_ref[...], ws_ref[...], preferred_element_type=jnp.float32)
    acc = acc + jnp.dot(hn, wn_ref[...], preferred_element_type=jnp.float32)
    out = jnp.maximum(acc + b_ref[...], 0.0)
    if readout:
        rid = pl.program_id(0) * BLK + lax.broadcasted_iota(jnp.int32, (BLK, 1), 0)
        out = jnp.where(rid < N, out, 0.0)
        o_ref[...] = jnp.sum(out, axis=0, keepdims=True)[None]
    else:
        o_ref[...] = out


def _combine(h, s_all, deg_col, Ws, Wn, b, readout=False):
    nb = GRID
    in_specs = [
        pl.BlockSpec((BLK, D), lambda i: (i, 0)),
        pl.BlockSpec((BLK, D), lambda i: (i, 0)),
        pl.BlockSpec((BLK, D), lambda i: (i + nb, 0)),
        pl.BlockSpec((BLK, 1), lambda i: (i, 0)),
        pl.BlockSpec((BLK, 1), lambda i: (i + nb, 0)),
        pl.BlockSpec((D, D), lambda i: (0, 0)),
        pl.BlockSpec((D, D), lambda i: (0, 0)),
        pl.BlockSpec((1, D), lambda i: (0, 0)),
    ]
    if readout:
        out_shape = jax.ShapeDtypeStruct((GRID, 1, D), jnp.float32)
        out_spec = pl.BlockSpec((1, 1, D), lambda i: (i, 0, 0))
    else:
        out_shape = jax.ShapeDtypeStruct((NPAD, D), jnp.float32)
        out_spec = pl.BlockSpec((BLK, D), lambda i: (i, 0))
    return pl.pallas_call(
        functools.partial(_combine_body, readout=readout),
        grid=(GRID,),
        in_specs=in_specs,
        out_specs=out_spec,
        out_shape=out_shape,
    )(h, s_all, s_all, deg_col, deg_col, Ws, Wn, b)


def _readout_body(ps_ref, wc_ref, bc_ref, o_ref):
    total = jnp.sum(ps_ref[...], axis=0) * (1.0 / N)
    o_ref[...] = jnp.dot(total, wc_ref[...],
                         preferred_element_type=jnp.float32) + bc_ref[...]


def _readout(parts, Wc, bc):
    return pl.pallas_call(
        _readout_body,
        out_shape=jax.ShapeDtypeStruct((1, NCLS), jnp.float32),
    )(parts, Wc, bc)


def kernel(x, edge_index, W_self0, W_neigh0, b0, W_self1, W_neigh1, b1,
           W_self2, W_neigh2, b2, W_cls, b_cls):
    src = edge_index[0].astype(jnp.int32)
    dst = edge_index[1].astype(jnp.int32)
    src_p = jnp.concatenate([src, jnp.zeros((EPAD - E,), jnp.int32)])
    dst_p = jnp.concatenate([dst, jnp.full((EPAD - E,), DUMMY, jnp.int32)])
    dst2d = dst_p.reshape(NCHUNKS, CHUNK)
    z2 = jnp.zeros((NPAD, D), jnp.float32)
    z1 = jnp.zeros((NPAD,), jnp.float32)

    h = jnp.pad(x, ((0, NPAD - N), (0, 0)))
    s_all, deg_all = _sc_agg_deg(h, src_p, dst2d, z2, z1)
    deg_col = deg_all.reshape(-1, 1)
    h = _combine(h, s_all, deg_col, W_self0, W_neigh0, b0.reshape(1, D))
    s_all = _sc_agg(h, src_p, dst2d, z2)
    h = _combine(h, s_all, deg_col, W_self1, W_neigh1, b1.reshape(1, D))
    s_all = _sc_agg(h, src_p, dst2d, z2)
    parts = _combine(h, s_all, deg_col, W_self2, W_neigh2, b2.reshape(1, D),
                     readout=True)
    return _readout(parts, W_cls, b_cls.reshape(1, NCLS))
